# baseline (speedup 1.0000x reference)
import functools

import jax
import jax.numpy as jnp
from jax.experimental import pallas as pl
from jax.experimental.pallas import tpu as pltpu


def _proj_body(ea_ref, we1_ref, we2_ref, p1_ref, p2_ref):
    ea = ea_ref[...]
    p1_ref[...] = ea @ we1_ref[...]
    p2_ref[...] = ea @ we2_ref[...]


def _edge_proj(ea, we1, we2, *, blk=16384):
    e, de = ea.shape
    d = we1.shape[1]
    row = lambda i: (i, 0)
    full = lambda i: (0, 0)
    return pl.pallas_call(
        _proj_body,
        grid=(e // blk,),
        in_specs=[
            pl.BlockSpec((blk, de), row),
            pl.BlockSpec((de, d), full),
            pl.BlockSpec((de, d), full),
        ],
        out_specs=[pl.BlockSpec((blk, d), row), pl.BlockSpec((blk, d), row)],
        out_shape=[jax.ShapeDtypeStruct((e, d), jnp.float32),
                   jax.ShapeDtypeStruct((e, d), jnp.float32)],
    )(ea, we1, we2)


def _gin_body(x_ref, agg_ref, wa_ref, ba_ref, wb_ref, bb_ref, o_ref, *,
              outer_relu):
    hh = x_ref[...] + agg_ref[...]
    h = jnp.maximum(hh @ wa_ref[...] + ba_ref[...], 0.0)
    o = h @ wb_ref[...] + bb_ref[...]
    if outer_relu:
        o = jnp.maximum(o, 0.0)
    o_ref[...] = o


def _gin_dense(x, agg, wa, ba, wb, bb, *, outer_relu, blk=2048):
    n, d = x.shape
    d2 = wa.shape[1]
    row = lambda i: (i, 0)
    full = lambda i: (0, 0)
    return pl.pallas_call(
        functools.partial(_gin_body, outer_relu=outer_relu),
        grid=(n // blk,),
        in_specs=[
            pl.BlockSpec((blk, d), row),
            pl.BlockSpec((blk, d), row),
            pl.BlockSpec((d, d2), full),
            pl.BlockSpec((1, d2), full),
            pl.BlockSpec((d2, d), full),
            pl.BlockSpec((1, d), full),
        ],
        out_specs=pl.BlockSpec((blk, d), row),
        out_shape=jax.ShapeDtypeStruct((n, d), jnp.float32),
    )(x, agg, wa, ba, wb, bb)


def _hw_body(h_ref, wmu_ref, ws_ref, hwm_ref, hws_ref):
    h = h_ref[...]
    hwm_ref[...] = h @ wmu_ref[...]
    hws_ref[...] = h @ ws_ref[...]


def _hw_dense(h, wmu, ws, *, blk=4096):
    n, d = h.shape
    row = lambda i: (i, 0)
    full = lambda i: (0, 0)
    return pl.pallas_call(
        _hw_body,
        grid=(n // blk,),
        in_specs=[
            pl.BlockSpec((blk, d), row),
            pl.BlockSpec((d, d), full),
            pl.BlockSpec((d, d), full),
        ],
        out_specs=[pl.BlockSpec((blk, d), row), pl.BlockSpec((blk, d), row)],
        out_shape=[jax.ShapeDtypeStruct((n, d), jnp.float32),
                   jax.ShapeDtypeStruct((n, d), jnp.float32)],
    )(h, wmu, ws)


def _final_body(aggm_ref, aggs_ref, hwm_ref, hws_ref, dinv_ref, bmu_ref,
                bs_ref, noise_ref, z_ref, adj_ref, *, blk):
    dinv = dinv_ref[...]
    d2 = dinv * dinv
    m = jnp.maximum((aggm_ref[...] + hwm_ref[...] * d2) + bmu_ref[...], 0.0)
    s = jnp.maximum((aggs_ref[...] + hws_ref[...] * d2) + bs_ref[...], 0.0)
    z = m + noise_ref[...] * jnp.exp(s)
    z_ref[...] = z
    d = z.shape[1]
    zb = z.reshape(blk // d, d, d)
    adj_ref[...] = jax.lax.dot_general(
        zb, zb, (((2,), (2,)), ((0,), (0,))),
        preferred_element_type=jnp.float32)


def _final_dense(aggm, aggs, hwm, hws, dinv, bmu, bs, noise, *, blk=1024):
    n, d = aggm.shape
    b = blk // d
    row = lambda i: (i, 0)
    full = lambda i: (0, 0)
    return pl.pallas_call(
        functools.partial(_final_body, blk=blk),
        grid=(n // blk,),
        in_specs=[
            pl.BlockSpec((blk, d), row),
            pl.BlockSpec((blk, d), row),
            pl.BlockSpec((blk, d), row),
            pl.BlockSpec((blk, d), row),
            pl.BlockSpec((blk, 1), row),
            pl.BlockSpec((1, d), full),
            pl.BlockSpec((1, d), full),
            pl.BlockSpec((blk, d), row),
        ],
        out_specs=[pl.BlockSpec((blk, d), row),
                   pl.BlockSpec((b, d, d), lambda i: (i, 0, 0))],
        out_shape=[jax.ShapeDtypeStruct((n, d), jnp.float32),
                   jax.ShapeDtypeStruct((n // d, d, d), jnp.float32)],
    )(aggm, aggs, hwm, hws, dinv, bmu, bs, noise)


def kernel(x, adj, edge_attr, We1, W1a, b1a, W1b, b1b, We2, W2a, b2a, W2b,
           b2b, Wmu, bmu, Ws, bs):
    n, d = x.shape
    e = edge_attr.shape[0]
    src = adj[0]
    dst = adj[1]

    b1a_ = b1a.reshape(1, -1)
    b1b_ = b1b.reshape(1, -1)
    b2a_ = b2a.reshape(1, -1)
    b2b_ = b2b.reshape(1, -1)
    bmu_ = bmu.reshape(1, -1)
    bs_ = bs.reshape(1, -1)

    proj1, proj2 = _edge_proj(edge_attr, We1, We2)

    agg1 = jax.ops.segment_sum(jnp.take(x, src, axis=0) + proj1, dst,
                               num_segments=n)
    h1 = _gin_dense(x, agg1, W1a, b1a_, W1b, b1b_, outer_relu=True)

    agg2 = jax.ops.segment_sum(jnp.take(h1, src, axis=0) + proj2, dst,
                               num_segments=n)
    h2 = _gin_dense(h1, agg2, W2a, b2a_, W2b, b2b_, outer_relu=False)

    deg = jax.ops.segment_sum(jnp.ones((e,), jnp.float32), dst,
                              num_segments=n) + 1.0
    dinv = 1.0 / jnp.sqrt(deg)

    hwm, hws = _hw_dense(h2, Wmu, Ws)
    coef = (dinv[src] * dinv[dst])[:, None]
    aggm = jax.ops.segment_sum(jnp.take(hwm, src, axis=0) * coef, dst,
                               num_segments=n)
    aggs = jax.ops.segment_sum(jnp.take(hws, src, axis=0) * coef, dst,
                               num_segments=n)

    noise = jax.random.normal(jax.random.key(42), (n, d), jnp.float32)
    z, adj_logits = _final_dense(aggm, aggs, hwm, hws, dinv.reshape(n, 1),
                                 bmu_, bs_, noise)
    return (z, adj_logits)


# speedup vs baseline: 2.7496x; 2.7496x over previous
import functools

import jax
import jax.numpy as jnp
from jax import lax
from jax.experimental import pallas as pl
from jax.experimental.pallas import tpu as pltpu
from jax.experimental.pallas import tpu_sc as plsc

_L = 16
_NS = 16
_NC = 2


_CH = 8192
_G = 512
_NW = 32
_KG = 2
_ZR = 16


def _make_segsum(n, e, with_proj, with_deg):
    ept = e // _NS
    nwin = ept // _NW
    ng = nwin // _KG
    bases = list(range(0, n, 2 * _CH))

    mesh = plsc.VectorSubcoreMesh(core_axis_name="c", subcore_axis_name="s")

    out_type = [jax.ShapeDtypeStruct((n, 128), jnp.float32)]
    if with_deg:
        out_type.append(jax.ShapeDtypeStruct((n,), jnp.float32))

    def slotset():
        return [
            [pltpu.VMEM((_NW, 128), jnp.float32) for _ in range(_KG)],
            [pltpu.VMEM((_NW,), jnp.int32) for _ in range(_KG)],
            [pltpu.VMEM((_NW,), jnp.int32) for _ in range(_KG)],
            [pltpu.VMEM((_NW,), jnp.int32) for _ in range(_KG)],
            [pltpu.SemaphoreType.DMA for _ in range(_KG)],
            [pltpu.SemaphoreType.DMA for _ in range(_KG)],
            [pltpu.SemaphoreType.DMA for _ in range(_KG)],
        ] + ([[pltpu.VMEM((_NW, 128), jnp.float32) for _ in range(_KG)]]
             if with_proj else [])

    flat_scratch = [
        pltpu.VMEM_SHARED((_CH + _G, 128), jnp.float32),
        pltpu.VMEM((_ZR, 128), jnp.float32),
        slotset(), slotset(), slotset(),
    ]
    if with_deg:
        flat_scratch += [
            pltpu.VMEM_SHARED((n,), jnp.float32),
            pltpu.VMEM((_NW,), jnp.float32),
            pltpu.VMEM((1024,), jnp.float32),
            pltpu.SemaphoreType.DMA,
        ]

    def body(tbl, srcv, dstv, *rest):
        it = iter(rest)
        proj = next(it) if with_proj else None
        out = next(it)
        deg = next(it) if with_deg else None
        acc = next(it)
        zbuf = next(it)
        sets = [next(it), next(it), next(it)]
        if with_deg:
            dacc = next(it)
            ones = next(it)
            zbuf1 = next(it)
            dsem = next(it)
        npt = n // _NS

        cid = lax.axis_index("c")
        tid = lax.axis_index("s")
        iota = jax.lax.iota(jnp.int32, _L)

        zero16 = jnp.zeros((_L,), jnp.float32)

        def zfill(j, _):
            for k in range(128 // _L):
                zbuf[j, pl.ds(k * _L, _L)] = zero16
            return 0
        lax.fori_loop(0, _ZR, zfill, 0)
        if with_deg:
            def zfill1(j, _):
                zbuf1[pl.ds(j * _L, _L)] = zero16
                return 0
            lax.fori_loop(0, 1024 // _L, zfill1, 0)
            one16 = jnp.full((_L,), 1.0, jnp.float32)
            for k in range(_NW // _L):
                ones[pl.ds(k * _L, _L)] = one16

        def issue_idx(g, st, do_deg):
            rows, sidx, didx, dstl, isem, gsem, ssem = st[:7]
            prows = st[7] if with_proj else None
            for b in range(_KG):
                w = g * _KG + b
                e0 = tid * ept + w * _NW
                pltpu.async_copy(srcv.at[pl.ds(e0, _NW)], sidx[b], isem[b])
                pltpu.async_copy(dstv.at[pl.ds(e0, _NW)], didx[b], isem[b])
                if with_proj:
                    pltpu.async_copy(proj.at[pl.ds(e0, _NW)], prows[b],
                                     isem[b])

        def issue_gather(g, st, base, ch, do_deg):
            rows, sidx, didx, dstl, isem, gsem, ssem = st[:7]
            prows = st[7] if with_proj else None
            for b in range(_KG):
                w = g * _KG + b
                e0 = tid * ept + w * _NW
                pltpu.make_async_copy(srcv.at[pl.ds(e0, _NW)], sidx[b],
                                      isem[b]).wait()
                pltpu.make_async_copy(dstv.at[pl.ds(e0, _NW)], didx[b],
                                      isem[b]).wait()
                if with_proj:
                    pltpu.make_async_copy(proj.at[pl.ds(e0, _NW)], prows[b],
                                          isem[b]).wait()
                for k in range(_NW // _L):
                    dv = didx[b][pl.ds(k * _L, _L)]
                    lv = dv - base
                    m = (lv >= 0) & (lv < ch)
                    gv = _CH + ((iota * 17 + k * 53 + w * 29) & (_G - 1))
                    dstl[b][pl.ds(k * _L, _L)] = jnp.where(m, lv, gv)
                pltpu.async_copy(tbl.at[sidx[b]], rows[b], gsem[b])

        def do_scatter(g, st, do_deg):
            rows, sidx, didx, dstl, isem, gsem, ssem = st[:7]
            prows = st[7] if with_proj else None
            for b in range(_KG):
                pltpu.make_async_copy(tbl.at[sidx[b]], rows[b],
                                      gsem[b]).wait()
                drow = dstl[b]
                pltpu.async_copy(rows[b], acc.at[drow], ssem[b], add=True)
                if with_proj:
                    pltpu.async_copy(prows[b], acc.at[drow], ssem[b],
                                     add=True)
                if do_deg:
                    @pl.when(cid == 0)
                    def _():
                        pltpu.async_copy(ones, dacc.at[didx[b]], dsem,
                                         add=True)

        def drain_s(st):
            rows, sidx, didx, dstl, isem, gsem, ssem = st[:7]
            prows = st[7] if with_proj else None
            for b in range(_KG):
                pltpu.make_async_copy(rows[b], acc.at[dstl[b]],
                                      ssem[b]).wait()
                if with_proj:
                    pltpu.make_async_copy(prows[b], acc.at[dstl[b]],
                                          ssem[b]).wait()

        for rr, gbase in enumerate(bases):
            ch = min(_CH, (n - gbase) // 2)
            base = gbase + cid * ch
            cpt = ch // _NS
            do_deg = with_deg and rr == 0

            for j in range(cpt // _ZR):
                pltpu.sync_copy(zbuf,
                                acc.at[pl.ds(tid * cpt + j * _ZR, _ZR)])
            if do_deg:
                @pl.when(cid == 0)
                def _():
                    for jz in range(npt // 1024):
                        pltpu.sync_copy(
                            zbuf1,
                            dacc.at[pl.ds(tid * npt + jz * 1024, 1024)])
            plsc.subcore_barrier()

            A, B, C = sets
            issue_idx(0, A, do_deg)
            issue_idx(1, B, do_deg)
            issue_idx(2, C, do_deg)
            issue_gather(0, A, base, ch, do_deg)
            issue_gather(1, B, base, ch, do_deg)

            def tri(i, _):
                g0 = 3 * i

                def step(goff, st, st_next2):
                    g = g0 + goff

                    @pl.when(g < ng)
                    def _():
                        do_scatter(g, st, do_deg)

                    @pl.when(g + 3 < ng)
                    def _():
                        drain_s(st)
                        issue_idx(g + 3, st, do_deg)

                    @pl.when(g + 2 < ng)
                    def _():
                        issue_gather(g + 2, st_next2, base, ch, do_deg)

                step(0, A, C)
                step(1, B, A)
                step(2, C, B)
                return 0

            lax.fori_loop(0, (ng + 2) // 3, tri, 0)
            drain_s(A)
            drain_s(B)
            drain_s(C)
            if do_deg:
                @pl.when(cid == 0)
                def _():
                    def dwait(j, _):
                        pltpu.make_async_copy(ones, dacc.at[sets[0][3][0]],
                                              dsem).wait()
                        return 0
                    lax.fori_loop(0, nwin, dwait, 0)
            plsc.subcore_barrier()

            pltpu.sync_copy(acc.at[pl.ds(tid * cpt, cpt)],
                            out.at[pl.ds(base + tid * cpt, cpt)])
            if do_deg:
                @pl.when(cid == 0)
                def _():
                    pltpu.sync_copy(dacc.at[pl.ds(tid * npt, npt)],
                                    deg.at[pl.ds(tid * npt, npt)])
            plsc.subcore_barrier()

    return pl.kernel(body, out_type=out_type, mesh=mesh,
                     scratch_types=flat_scratch)


def _segsum_sc(table, src, dst, proj=None, with_deg=False):
    n, d = table.shape
    e = src.shape[0]
    k = _make_segsum(n, e, proj is not None, with_deg)
    args = [table, src, dst] + ([proj] if proj is not None else [])
    res = k(*args)
    return res if with_deg else res[0]


def _proj_body(ea_ref, we1_ref, we2_ref, p1_ref, p2_ref):
    ea = ea_ref[...]
    p1_ref[...] = ea @ we1_ref[...]
    p2_ref[...] = ea @ we2_ref[...]


def _edge_proj(ea, we1, we2, *, blk=16384):
    e, de = ea.shape
    d = we1.shape[1]
    row = lambda i: (i, 0)
    full = lambda i: (0, 0)
    return pl.pallas_call(
        _proj_body,
        grid=(e // blk,),
        in_specs=[
            pl.BlockSpec((blk, de), row),
            pl.BlockSpec((de, d), full),
            pl.BlockSpec((de, d), full),
        ],
        out_specs=[pl.BlockSpec((blk, d), row), pl.BlockSpec((blk, d), row)],
        out_shape=[jax.ShapeDtypeStruct((e, d), jnp.float32),
                   jax.ShapeDtypeStruct((e, d), jnp.float32)],
    )(ea, we1, we2)


def _gin_body(x_ref, agg_ref, wa_ref, ba_ref, wb_ref, bb_ref, o_ref, *,
              outer_relu):
    hh = x_ref[...] + agg_ref[...]
    h = jnp.maximum(hh @ wa_ref[...] + ba_ref[...], 0.0)
    o = h @ wb_ref[...] + bb_ref[...]
    if outer_relu:
        o = jnp.maximum(o, 0.0)
    o_ref[...] = o


def _gin_dense(x, agg, wa, ba, wb, bb, *, outer_relu, blk=2048):
    n, d = x.shape
    d2 = wa.shape[1]
    row = lambda i: (i, 0)
    full = lambda i: (0, 0)
    return pl.pallas_call(
        functools.partial(_gin_body, outer_relu=outer_relu),
        grid=(n // blk,),
        in_specs=[
            pl.BlockSpec((blk, d), row),
            pl.BlockSpec((blk, d), row),
            pl.BlockSpec((d, d2), full),
            pl.BlockSpec((1, d2), full),
            pl.BlockSpec((d2, d), full),
            pl.BlockSpec((1, d), full),
        ],
        out_specs=pl.BlockSpec((blk, d), row),
        out_shape=jax.ShapeDtypeStruct((n, d), jnp.float32),
    )(x, agg, wa, ba, wb, bb)


def _hw_body(h_ref, deg_ref, wmu_ref, ws_ref, hwm_ref, hws_ref, tm_ref,
             ts_ref):
    h = h_ref[...]
    dinv = 1.0 / jnp.sqrt(deg_ref[...] + 1.0)
    hwm = h @ wmu_ref[...]
    hws = h @ ws_ref[...]
    hwm_ref[...] = hwm
    hws_ref[...] = hws
    tm_ref[...] = hwm * dinv
    ts_ref[...] = hws * dinv


def _hw_dense(h, deg, wmu, ws, *, blk=4096):
    n, d = h.shape
    row = lambda i: (i, 0)
    full = lambda i: (0, 0)
    return pl.pallas_call(
        _hw_body,
        grid=(n // blk,),
        in_specs=[
            pl.BlockSpec((blk, d), row),
            pl.BlockSpec((blk, 1), row),
            pl.BlockSpec((d, d), full),
            pl.BlockSpec((d, d), full),
        ],
        out_specs=[pl.BlockSpec((blk, d), row) for _ in range(4)],
        out_shape=[jax.ShapeDtypeStruct((n, d), jnp.float32)
                   for _ in range(4)],
    )(h, deg, wmu, ws)


def _final_body(aggm_ref, aggs_ref, hwm_ref, hws_ref, deg_ref, bmu_ref,
                bs_ref, noise_ref, z_ref, adj_ref, *, blk):
    d = hwm_ref.shape[1]
    dinv = 1.0 / jnp.sqrt(deg_ref[...] + 1.0)
    d2 = dinv * dinv
    aggm = aggm_ref[...] * dinv
    aggs = aggs_ref[...] * dinv
    m = jnp.maximum((aggm + hwm_ref[...] * d2) + bmu_ref[...], 0.0)
    s = jnp.maximum((aggs + hws_ref[...] * d2) + bs_ref[...], 0.0)
    z = m + noise_ref[...] * jnp.exp(s)
    z_ref[...] = z
    zb = z.reshape(blk // d, d, d)
    adj_ref[...] = jax.lax.dot_general(
        zb, zb, (((2,), (2,)), ((0,), (0,))),
        preferred_element_type=jnp.float32)


def _final_dense(aggm, aggs, hwm, hws, deg, bmu, bs, noise, *, blk=1024):
    n, d = hwm.shape
    b = blk // d
    row = lambda i: (i, 0)
    full = lambda i: (0, 0)
    return pl.pallas_call(
        functools.partial(_final_body, blk=blk),
        grid=(n // blk,),
        in_specs=[
            pl.BlockSpec((blk, d), row),
            pl.BlockSpec((blk, d), row),
            pl.BlockSpec((blk, d), row),
            pl.BlockSpec((blk, d), row),
            pl.BlockSpec((blk, 1), row),
            pl.BlockSpec((1, d), full),
            pl.BlockSpec((1, d), full),
            pl.BlockSpec((blk, d), row),
        ],
        out_specs=[pl.BlockSpec((blk, d), row),
                   pl.BlockSpec((b, d, d), lambda i: (i, 0, 0))],
        out_shape=[jax.ShapeDtypeStruct((n, d), jnp.float32),
                   jax.ShapeDtypeStruct((n // d, d, d), jnp.float32)],
    )(aggm, aggs, hwm, hws, deg, bmu, bs, noise)


def kernel(x, adj, edge_attr, We1, W1a, b1a, W1b, b1b, We2, W2a, b2a, W2b,
           b2b, Wmu, bmu, Ws, bs):
    n, d = x.shape
    e = edge_attr.shape[0]
    srcv = adj[0]
    dstv = adj[1]

    b1a_ = b1a.reshape(1, -1)
    b1b_ = b1b.reshape(1, -1)
    b2a_ = b2a.reshape(1, -1)
    b2b_ = b2b.reshape(1, -1)
    bmu_ = bmu.reshape(1, -1)
    bs_ = bs.reshape(1, -1)

    proj1, proj2 = _edge_proj(edge_attr, We1, We2)

    agg1, deg = _segsum_sc(x, srcv, dstv, proj1, with_deg=True)
    h1 = _gin_dense(x, agg1, W1a, b1a_, W1b, b1b_, outer_relu=True)

    agg2 = _segsum_sc(h1, srcv, dstv, proj2)
    h2 = _gin_dense(h1, agg2, W2a, b2a_, W2b, b2b_, outer_relu=False)

    deg_ = deg.reshape(n, 1)
    hwm, hws, tabm, tabs = _hw_dense(h2, deg_, Wmu, Ws)

    aggm = _segsum_sc(tabm, srcv, dstv)
    aggs = _segsum_sc(tabs, srcv, dstv)

    noise = jax.random.normal(jax.random.key(42), (n, d), jnp.float32)
    z, adj_logits = _final_dense(aggm, aggs, hwm, hws, deg_, bmu_, bs_,
                                 noise)
    return (z, adj_logits)


# speedup vs baseline: 3.0067x; 1.0935x over previous
import functools

import jax
import jax.numpy as jnp
from jax import lax
from jax.experimental import pallas as pl
from jax.experimental.pallas import tpu as pltpu
from jax.experimental.pallas import tpu_sc as plsc

_L = 16
_NS = 16
_NC = 2


_CH = 8192
_G = 512
_NW = 32
_KG = 2
_ZR = 16


def _make_segsum(n, e, with_proj, with_deg):
    ch_rows = _CH if with_proj else 11776
    ept = e // _NS
    nwin = ept // _NW
    ng = nwin // _KG
    bases = list(range(0, n, 2 * ch_rows))

    mesh = plsc.VectorSubcoreMesh(core_axis_name="c", subcore_axis_name="s")

    out_type = [jax.ShapeDtypeStruct((n, 128), jnp.float32)]
    if with_deg:
        out_type.append(jax.ShapeDtypeStruct((n,), jnp.float32))

    def slotset():
        return [
            [pltpu.VMEM((_NW, 128), jnp.float32) for _ in range(_KG)],
            [pltpu.VMEM((_NW,), jnp.int32) for _ in range(_KG)],
            [pltpu.VMEM((_NW,), jnp.int32) for _ in range(_KG)],
            [pltpu.VMEM((_NW,), jnp.int32) for _ in range(_KG)],
            [pltpu.SemaphoreType.DMA for _ in range(_KG)],
            [pltpu.SemaphoreType.DMA for _ in range(_KG)],
            [pltpu.SemaphoreType.DMA for _ in range(_KG)],
        ] + ([[pltpu.VMEM((_NW, 128), jnp.float32) for _ in range(_KG)]]
             if with_proj else [])

    flat_scratch = [
        pltpu.VMEM_SHARED((ch_rows + _G, 128), jnp.float32),
        pltpu.VMEM((_ZR, 128), jnp.float32),
        slotset(), slotset(), slotset(),
    ]
    if with_deg:
        flat_scratch += [
            pltpu.VMEM_SHARED((n,), jnp.float32),
            pltpu.VMEM((_NW,), jnp.float32),
            pltpu.VMEM((1024,), jnp.float32),
            pltpu.SemaphoreType.DMA,
        ]

    def body(tbl, srcv, dstv, *rest):
        it = iter(rest)
        proj = next(it) if with_proj else None
        out = next(it)
        deg = next(it) if with_deg else None
        acc = next(it)
        zbuf = next(it)
        sets = [next(it), next(it), next(it)]
        if with_deg:
            dacc = next(it)
            ones = next(it)
            zbuf1 = next(it)
            dsem = next(it)
        npt = n // _NS

        cid = lax.axis_index("c")
        tid = lax.axis_index("s")
        iota = jax.lax.iota(jnp.int32, _L)

        zero16 = jnp.zeros((_L,), jnp.float32)

        def zfill(j, _):
            for k in range(128 // _L):
                zbuf[j, pl.ds(k * _L, _L)] = zero16
            return 0
        lax.fori_loop(0, _ZR, zfill, 0)
        if with_deg:
            def zfill1(j, _):
                zbuf1[pl.ds(j * _L, _L)] = zero16
                return 0
            lax.fori_loop(0, 1024 // _L, zfill1, 0)
            one16 = jnp.full((_L,), 1.0, jnp.float32)
            for k in range(_NW // _L):
                ones[pl.ds(k * _L, _L)] = one16

        def issue_idx(g, st, do_deg):
            rows, sidx, didx, dstl, isem, gsem, ssem = st[:7]
            prows = st[7] if with_proj else None
            for b in range(_KG):
                w = g * _KG + b
                e0 = tid * ept + w * _NW
                pltpu.async_copy(srcv.at[pl.ds(e0, _NW)], sidx[b], isem[b])
                pltpu.async_copy(dstv.at[pl.ds(e0, _NW)], didx[b], isem[b])
                if with_proj:
                    pltpu.async_copy(proj.at[pl.ds(e0, _NW)], prows[b],
                                     isem[b])

        def issue_gather(g, st, base, ch, do_deg):
            rows, sidx, didx, dstl, isem, gsem, ssem = st[:7]
            prows = st[7] if with_proj else None
            for b in range(_KG):
                w = g * _KG + b
                e0 = tid * ept + w * _NW
                pltpu.make_async_copy(srcv.at[pl.ds(e0, _NW)], sidx[b],
                                      isem[b]).wait()
                pltpu.make_async_copy(dstv.at[pl.ds(e0, _NW)], didx[b],
                                      isem[b]).wait()
                if with_proj:
                    pltpu.make_async_copy(proj.at[pl.ds(e0, _NW)], prows[b],
                                          isem[b]).wait()
                for k in range(_NW // _L):
                    dv = didx[b][pl.ds(k * _L, _L)]
                    lv = dv - base
                    m = (lv >= 0) & (lv < ch)
                    gv = ch_rows + ((iota * 17 + k * 53 + w * 29) & (_G - 1))
                    dstl[b][pl.ds(k * _L, _L)] = jnp.where(m, lv, gv)
                pltpu.async_copy(tbl.at[sidx[b]], rows[b], gsem[b])

        def do_scatter(g, st, do_deg):
            rows, sidx, didx, dstl, isem, gsem, ssem = st[:7]
            prows = st[7] if with_proj else None
            for b in range(_KG):
                pltpu.make_async_copy(tbl.at[sidx[b]], rows[b],
                                      gsem[b]).wait()
                drow = dstl[b]
                pltpu.async_copy(rows[b], acc.at[drow], ssem[b], add=True)
                if with_proj:
                    pltpu.async_copy(prows[b], acc.at[drow], ssem[b],
                                     add=True)
                if do_deg:
                    @pl.when(cid == 0)
                    def _():
                        pltpu.async_copy(ones, dacc.at[didx[b]], dsem,
                                         add=True)

        def drain_s(st):
            rows, sidx, didx, dstl, isem, gsem, ssem = st[:7]
            prows = st[7] if with_proj else None
            for b in range(_KG):
                pltpu.make_async_copy(rows[b], acc.at[dstl[b]],
                                      ssem[b]).wait()
                if with_proj:
                    pltpu.make_async_copy(prows[b], acc.at[dstl[b]],
                                          ssem[b]).wait()

        for rr, gbase in enumerate(bases):
            ch = min(ch_rows, (n - gbase) // 2)
            base = gbase + cid * ch
            cpt = ch // _NS
            do_deg = with_deg and rr == 0

            for j in range(cpt // _ZR):
                pltpu.sync_copy(zbuf,
                                acc.at[pl.ds(tid * cpt + j * _ZR, _ZR)])
            if do_deg:
                @pl.when(cid == 0)
                def _():
                    for jz in range(npt // 1024):
                        pltpu.sync_copy(
                            zbuf1,
                            dacc.at[pl.ds(tid * npt + jz * 1024, 1024)])
            plsc.subcore_barrier()

            A, B, C = sets
            issue_idx(0, A, do_deg)
            issue_idx(1, B, do_deg)
            issue_idx(2, C, do_deg)
            issue_gather(0, A, base, ch, do_deg)
            issue_gather(1, B, base, ch, do_deg)

            def tri(i, _):
                g0 = 3 * i

                def step(goff, st, st_next2):
                    g = g0 + goff

                    @pl.when(g < ng)
                    def _():
                        do_scatter(g, st, do_deg)

                    @pl.when(g + 3 < ng)
                    def _():
                        drain_s(st)
                        issue_idx(g + 3, st, do_deg)

                    @pl.when(g + 2 < ng)
                    def _():
                        issue_gather(g + 2, st_next2, base, ch, do_deg)

                step(0, A, C)
                step(1, B, A)
                step(2, C, B)
                return 0

            lax.fori_loop(0, (ng + 2) // 3, tri, 0)
            drain_s(A)
            drain_s(B)
            drain_s(C)
            if do_deg:
                @pl.when(cid == 0)
                def _():
                    def dwait(j, _):
                        pltpu.make_async_copy(ones, dacc.at[sets[0][3][0]],
                                              dsem).wait()
                        return 0
                    lax.fori_loop(0, nwin, dwait, 0)
            plsc.subcore_barrier()

            pltpu.sync_copy(acc.at[pl.ds(tid * cpt, cpt)],
                            out.at[pl.ds(base + tid * cpt, cpt)])
            if do_deg:
                @pl.when(cid == 0)
                def _():
                    pltpu.sync_copy(dacc.at[pl.ds(tid * npt, npt)],
                                    deg.at[pl.ds(tid * npt, npt)])
            plsc.subcore_barrier()

    return pl.kernel(body, out_type=out_type, mesh=mesh,
                     scratch_types=flat_scratch)


def _segsum_sc(table, src, dst, proj=None, with_deg=False):
    n, d = table.shape
    e = src.shape[0]
    k = _make_segsum(n, e, proj is not None, with_deg)
    args = [table, src, dst] + ([proj] if proj is not None else [])
    res = k(*args)
    return res if with_deg else res[0]


def _proj_body(ea_ref, we1_ref, we2_ref, p1_ref, p2_ref):
    ea = ea_ref[...]
    p1_ref[...] = ea @ we1_ref[...]
    p2_ref[...] = ea @ we2_ref[...]


def _edge_proj(ea, we1, we2, *, blk=16384):
    e, de = ea.shape
    d = we1.shape[1]
    row = lambda i: (i, 0)
    full = lambda i: (0, 0)
    return pl.pallas_call(
        _proj_body,
        grid=(e // blk,),
        in_specs=[
            pl.BlockSpec((blk, de), row),
            pl.BlockSpec((de, d), full),
            pl.BlockSpec((de, d), full),
        ],
        out_specs=[pl.BlockSpec((blk, d), row), pl.BlockSpec((blk, d), row)],
        out_shape=[jax.ShapeDtypeStruct((e, d), jnp.float32),
                   jax.ShapeDtypeStruct((e, d), jnp.float32)],
    )(ea, we1, we2)


def _gin_body(x_ref, agg_ref, wa_ref, ba_ref, wb_ref, bb_ref, o_ref, *,
              outer_relu):
    hh = x_ref[...] + agg_ref[...]
    h = jnp.maximum(hh @ wa_ref[...] + ba_ref[...], 0.0)
    o = h @ wb_ref[...] + bb_ref[...]
    if outer_relu:
        o = jnp.maximum(o, 0.0)
    o_ref[...] = o


def _gin_dense(x, agg, wa, ba, wb, bb, *, outer_relu, blk=2048):
    n, d = x.shape
    d2 = wa.shape[1]
    row = lambda i: (i, 0)
    full = lambda i: (0, 0)
    return pl.pallas_call(
        functools.partial(_gin_body, outer_relu=outer_relu),
        grid=(n // blk,),
        in_specs=[
            pl.BlockSpec((blk, d), row),
            pl.BlockSpec((blk, d), row),
            pl.BlockSpec((d, d2), full),
            pl.BlockSpec((1, d2), full),
            pl.BlockSpec((d2, d), full),
            pl.BlockSpec((1, d), full),
        ],
        out_specs=pl.BlockSpec((blk, d), row),
        out_shape=jax.ShapeDtypeStruct((n, d), jnp.float32),
    )(x, agg, wa, ba, wb, bb)


def _hw_body(h_ref, deg_ref, wmu_ref, ws_ref, hwm_ref, hws_ref, tm_ref,
             ts_ref):
    h = h_ref[...]
    dinv = 1.0 / jnp.sqrt(deg_ref[...] + 1.0)
    hwm = h @ wmu_ref[...]
    hws = h @ ws_ref[...]
    hwm_ref[...] = hwm
    hws_ref[...] = hws
    tm_ref[...] = hwm * dinv
    ts_ref[...] = hws * dinv


def _hw_dense(h, deg, wmu, ws, *, blk=4096):
    n, d = h.shape
    row = lambda i: (i, 0)
    full = lambda i: (0, 0)
    return pl.pallas_call(
        _hw_body,
        grid=(n // blk,),
        in_specs=[
            pl.BlockSpec((blk, d), row),
            pl.BlockSpec((blk, 1), row),
            pl.BlockSpec((d, d), full),
            pl.BlockSpec((d, d), full),
        ],
        out_specs=[pl.BlockSpec((blk, d), row) for _ in range(4)],
        out_shape=[jax.ShapeDtypeStruct((n, d), jnp.float32)
                   for _ in range(4)],
    )(h, deg, wmu, ws)


def _final_body(aggm_ref, aggs_ref, hwm_ref, hws_ref, deg_ref, bmu_ref,
                bs_ref, noise_ref, z_ref, adj_ref, *, blk):
    d = hwm_ref.shape[1]
    dinv = 1.0 / jnp.sqrt(deg_ref[...] + 1.0)
    d2 = dinv * dinv
    aggm = aggm_ref[...] * dinv
    aggs = aggs_ref[...] * dinv
    m = jnp.maximum((aggm + hwm_ref[...] * d2) + bmu_ref[...], 0.0)
    s = jnp.maximum((aggs + hws_ref[...] * d2) + bs_ref[...], 0.0)
    z = m + noise_ref[...] * jnp.exp(s)
    z_ref[...] = z
    zb = z.reshape(blk // d, d, d)
    adj_ref[...] = jax.lax.dot_general(
        zb, zb, (((2,), (2,)), ((0,), (0,))),
        preferred_element_type=jnp.float32)


def _final_dense(aggm, aggs, hwm, hws, deg, bmu, bs, noise, *, blk=1024):
    n, d = hwm.shape
    b = blk // d
    row = lambda i: (i, 0)
    full = lambda i: (0, 0)
    return pl.pallas_call(
        functools.partial(_final_body, blk=blk),
        grid=(n // blk,),
        in_specs=[
            pl.BlockSpec((blk, d), row),
            pl.BlockSpec((blk, d), row),
            pl.BlockSpec((blk, d), row),
            pl.BlockSpec((blk, d), row),
            pl.BlockSpec((blk, 1), row),
            pl.BlockSpec((1, d), full),
            pl.BlockSpec((1, d), full),
            pl.BlockSpec((blk, d), row),
        ],
        out_specs=[pl.BlockSpec((blk, d), row),
                   pl.BlockSpec((b, d, d), lambda i: (i, 0, 0))],
        out_shape=[jax.ShapeDtypeStruct((n, d), jnp.float32),
                   jax.ShapeDtypeStruct((n // d, d, d), jnp.float32)],
    )(aggm, aggs, hwm, hws, deg, bmu, bs, noise)


def kernel(x, adj, edge_attr, We1, W1a, b1a, W1b, b1b, We2, W2a, b2a, W2b,
           b2b, Wmu, bmu, Ws, bs):
    n, d = x.shape
    e = edge_attr.shape[0]
    srcv = adj[0]
    dstv = adj[1]

    b1a_ = b1a.reshape(1, -1)
    b1b_ = b1b.reshape(1, -1)
    b2a_ = b2a.reshape(1, -1)
    b2b_ = b2b.reshape(1, -1)
    bmu_ = bmu.reshape(1, -1)
    bs_ = bs.reshape(1, -1)

    proj1, proj2 = _edge_proj(edge_attr, We1, We2)

    agg1, deg = _segsum_sc(x, srcv, dstv, proj1, with_deg=True)
    h1 = _gin_dense(x, agg1, W1a, b1a_, W1b, b1b_, outer_relu=True)

    agg2 = _segsum_sc(h1, srcv, dstv, proj2)
    h2 = _gin_dense(h1, agg2, W2a, b2a_, W2b, b2b_, outer_relu=False)

    deg_ = deg.reshape(n, 1)
    hwm, hws, tabm, tabs = _hw_dense(h2, deg_, Wmu, Ws)

    aggm = _segsum_sc(tabm, srcv, dstv)
    aggs = _segsum_sc(tabs, srcv, dstv)

    noise = jax.random.normal(jax.random.key(42), (n, d), jnp.float32)
    z, adj_logits = _final_dense(aggm, aggs, hwm, hws, deg_, bmu_, bs_,
                                 noise)
    return (z, adj_logits)
